# baseline (speedup 1.0000x reference)
import dataclasses

import jax
import jax.numpy as jnp
import numpy as np
from jax import lax
from jax.experimental import pallas as pl
from jax.experimental.pallas import tpu as pltpu
from jax.experimental.pallas import tpu_sc as plsc

K = 64
ROWS = 64
COLS = 32768
CAP = 512
CAP2 = 128
NVREG = COLS // 16
NC = 2
NS = 16
ROWS_PER_W = ROWS // (NC * NS)
MININT = np.int32(-2147483648)
BIG = np.int32(2**31 - 1)


def _mono(v):
    b = plsc.bitcast(v, jnp.int32)
    return b ^ (lax.shift_right_arithmetic(b, 31) | MININT)


def _mono_u(v):
    return plsc.bitcast(_mono(v), jnp.uint32)


def _find_cross(h, acc, need):
    rh = lax.rev(h, (0,))
    cs = plsc.cumsum(rh)
    cond = (cs + acc) >= need
    j = jnp.max(plsc.all_reduce_ffs(cond))
    lane = 15 - j
    iota = lax.iota(jnp.int32, 16)
    above = acc + jnp.sum(jnp.where(iota == jnp.broadcast_to(j, (16,)),
                                    cs - rh, 0))
    return lane, above


def _select_bin(hist_v, totals_v, need):
    iota16 = lax.iota(jnp.int32, 16)
    lane0 = iota16 == 0

    @plsc.parallel_loop(0, 256, unroll=4)
    def _(c):
        h = hist_v[pl.ds(c * 16, 16)]
        tot = jnp.broadcast_to(jnp.sum(h), (16,))
        plsc.store_scatter(totals_v, [jnp.broadcast_to(c, (16,))], tot,
                           mask=lane0)

    def cscan(j, carry):
        acc, kstar, found = carry
        kk = 15 - j
        s = jnp.sum(totals_v[pl.ds(kk * 16, 16)])
        cond = jnp.logical_and(jnp.logical_not(found), (acc + s) >= need)
        kstar = jnp.where(cond, kk, kstar)
        acc = jnp.where(jnp.logical_or(cond, found), acc, acc + s)
        found = jnp.logical_or(found, cond)
        return acc, kstar, found

    acc0, kstar, _ = lax.fori_loop(
        0, 16, cscan, (jnp.int32(0), jnp.int32(0), False))

    tv = totals_v[pl.ds(kstar * 16, 16)]
    lane_c, acc1 = _find_cross(tv, acc0, need)
    cstar = kstar * 16 + lane_c

    hf = hist_v[pl.ds(cstar * 16, 16)]
    lane_f, above = _find_cross(hf, acc1, need)
    return cstar * 16 + lane_f, above


def _sc_body(x_hbm, vals_hbm, idx_hbm,
             row0_v, row1_v, hist_v, totals_v, cm_v, ci_v, c2m_v, c2i_v,
             vout_v, iout_v, sem0, sem1):
    wid = lax.axis_index("s") * NC + lax.axis_index("c")
    r0 = wid * ROWS_PER_W

    QTR = COLS // 4
    cps = [
        [pltpu.async_copy(x_hbm.at[r0 + rr, pl.ds(c * QTR, QTR)],
                          (row0_v if rr == 0 else row1_v).at[
                              pl.ds(c * QTR, QTR)],
                          sem0 if rr == 0 else sem1)
         for c in range(4)]
        for rr in range(ROWS_PER_W)
    ]

    iota16 = lax.iota(jnp.int32, 16)
    lane0 = iota16 == 0
    ones = jnp.ones((16,), jnp.int32)
    zeros = jnp.zeros((16,), jnp.int32)

    for rr in range(ROWS_PER_W):
        row_v = row0_v if rr == 0 else row1_v

        @plsc.parallel_loop(0, 256, unroll=8)
        def _(i):
            hist_v[pl.ds(i * 16, 16)] = zeros

        @plsc.parallel_loop(0, CAP // 16, unroll=8)
        def _(i):
            cm_v[pl.ds(i * 16, 16)] = zeros
            ci_v[pl.ds(i * 16, 16)] = zeros

        @plsc.parallel_loop(0, CAP2 // 16)
        def _(i):
            c2m_v[pl.ds(i * 16, 16)] = zeros
            c2i_v[pl.ds(i * 16, 16)] = zeros

        for c in range(4):
            cps[rr][c].wait()

            @plsc.parallel_loop(c * (NVREG // 4), (c + 1) * (NVREG // 4),
                                unroll=16)
            def _(i):
                mu = _mono_u(row_v[pl.ds(i * 16, 16)])
                binf = plsc.bitcast(mu >> jnp.uint32(20), jnp.int32)
                plsc.addupdate_scatter(hist_v, [binf], ones)

        bstar, above1 = _select_bin(hist_v, totals_v, K)
        tlo = jnp.broadcast_to(lax.shift_left(bstar, 20), (16,))
        tlo_u = plsc.bitcast(tlo, jnp.uint32)

        @plsc.parallel_loop(0, NVREG, unroll=16, carry=zeros)
        def p2(i, off_v):
            mu = _mono_u(row_v[pl.ds(i * 16, 16)])
            m = mu >= tlo_u
            pos = off_v + plsc.cumsum(m.astype(jnp.int32)) - 1
            pos = jnp.minimum(pos, jnp.int32(CAP - 1))
            idxv = iota16 + jnp.broadcast_to(i * 16, (16,))
            plsc.store_scatter(cm_v, [pos], plsc.bitcast(mu, jnp.int32),
                               mask=m)
            plsc.store_scatter(ci_v, [pos], idxv, mask=m)
            return off_v + plsc.all_reduce_population_count(m)

        @plsc.parallel_loop(0, 256, unroll=8)
        def _(i):
            hist_v[pl.ds(i * 16, 16)] = zeros

        bstar_u = plsc.bitcast(jnp.broadcast_to(bstar, (16,)), jnp.uint32)

        @plsc.parallel_loop(0, CAP // 16, unroll=4)
        def _(i):
            mu = plsc.bitcast(cm_v[pl.ds(i * 16, 16)], jnp.uint32)
            iseq = (mu >> jnp.uint32(20)) == bstar_u
            bin2 = plsc.bitcast((mu >> jnp.uint32(8)), jnp.int32) & 0xFFF
            plsc.addupdate_scatter(hist_v, [bin2], ones, mask=iseq)

        b2star, _ = _select_bin(hist_v, totals_v, K - above1)
        tlo2 = jnp.broadcast_to(
            lax.shift_left(bstar, 20) | lax.shift_left(b2star, 8), (16,))
        tlo2_u = plsc.bitcast(tlo2, jnp.uint32)

        @plsc.parallel_loop(0, CAP // 16, unroll=4, carry=zeros)
        def p3(i, off_v):
            mono = cm_v[pl.ds(i * 16, 16)]
            mu = plsc.bitcast(mono, jnp.uint32)
            m = mu >= tlo2_u
            pos = off_v + plsc.cumsum(m.astype(jnp.int32)) - 1
            pos = jnp.minimum(pos, jnp.int32(CAP2 - 1))
            plsc.store_scatter(c2m_v, [pos], mono, mask=m)
            plsc.store_scatter(c2i_v, [pos], ci_v[pl.ds(i * 16, 16)], mask=m)
            return off_v + plsc.all_reduce_population_count(m)

        NV2 = CAP2 // 16
        carry0 = tuple(c2m_v[pl.ds(j * 16, 16)] ^ MININT for j in range(NV2))

        @pl.loop(0, K, init_carry=carry0)
        def _(t, vs):
            m01 = jnp.maximum(vs[0], vs[1])
            m23 = jnp.maximum(vs[2], vs[3])
            m45 = jnp.maximum(vs[4], vs[5])
            m67 = jnp.maximum(vs[6], vs[7])
            mx = jnp.maximum(jnp.maximum(m01, m23), jnp.maximum(m45, m67))
            mxs = jnp.broadcast_to(jnp.max(mx), (16,))
            pos = jnp.full((16,), BIG, jnp.int32)
            for j in range(NV2):
                f = plsc.all_reduce_ffs(vs[j] == mxs)
                pj = jnp.where(f < 16, f + jnp.int32(j * 16), BIG)
                pos = jnp.minimum(pos, pj)
            idx_sel = plsc.load_gather(c2i_v, [pos])
            mono_sel = mxs ^ MININT
            dec = mono_sel ^ (~lax.shift_right_arithmetic(mono_sel, 31)
                             | MININT)
            tb = jnp.broadcast_to(t, (16,))
            plsc.store_scatter(vout_v, [tb],
                               plsc.bitcast(dec, jnp.float32), mask=lane0)
            plsc.store_scatter(iout_v, [tb], idx_sel, mask=lane0)
            new_vs = tuple(
                jnp.where((iota16 + jnp.int32(j * 16)) == pos, MININT, vs[j])
                for j in range(NV2))
            return new_vs

        pltpu.sync_copy(vout_v, vals_hbm.at[r0 + rr])
        pltpu.sync_copy(iout_v, idx_hbm.at[r0 + rr])


def kernel(x):
    mesh = plsc.VectorSubcoreMesh(
        core_axis_name="c", subcore_axis_name="s", num_cores=NC,
        num_subcores=NS)
    cp = pltpu.CompilerParams()
    if "needs_layout_passes" in pltpu.CompilerParams.__dataclass_fields__:
        cp = dataclasses.replace(cp, needs_layout_passes=False)
    kern = pl.kernel(
        _sc_body,
        out_type=[
            jax.ShapeDtypeStruct((ROWS, K), jnp.float32),
            jax.ShapeDtypeStruct((ROWS, K), jnp.int32),
        ],
        mesh=mesh,
        scratch_types=[
            pltpu.VMEM((COLS,), jnp.float32),
            pltpu.VMEM((COLS,), jnp.float32),
            pltpu.VMEM((4096,), jnp.int32),
            pltpu.VMEM((256,), jnp.int32),
            pltpu.VMEM((CAP,), jnp.int32),
            pltpu.VMEM((CAP,), jnp.int32),
            pltpu.VMEM((CAP2,), jnp.int32),
            pltpu.VMEM((CAP2,), jnp.int32),
            pltpu.VMEM((K,), jnp.float32),
            pltpu.VMEM((K,), jnp.int32),
            pltpu.SemaphoreType.DMA,
            pltpu.SemaphoreType.DMA,
        ],
        compiler_params=cp,
    )
    vals, idx = kern(x)
    return vals, idx


# speedup vs baseline: 1.0015x; 1.0015x over previous
import dataclasses

import jax
import jax.numpy as jnp
import numpy as np
from jax import lax
from jax.experimental import pallas as pl
from jax.experimental.pallas import tpu as pltpu
from jax.experimental.pallas import tpu_sc as plsc

K = 64
ROWS = 64
COLS = 32768
CAP = 512
CAP2 = 128
NVREG = COLS // 16
NC = 2
NS = 16
ROWS_PER_W = ROWS // (NC * NS)
MININT = np.int32(-2147483648)
BIG = np.int32(2**31 - 1)


def _mono(v):
    b = plsc.bitcast(v, jnp.int32)
    return b ^ (lax.shift_right_arithmetic(b, 31) | MININT)


def _mono_u(v):
    return plsc.bitcast(_mono(v), jnp.uint32)


def _find_cross(h, acc, need):
    rh = lax.rev(h, (0,))
    cs = plsc.cumsum(rh)
    cond = (cs + acc) >= need
    j = jnp.max(plsc.all_reduce_ffs(cond))
    lane = 15 - j
    iota = lax.iota(jnp.int32, 16)
    above = acc + jnp.sum(jnp.where(iota == jnp.broadcast_to(j, (16,)),
                                    cs - rh, 0))
    return lane, above


def _select_bin(hist_v, totals_v, need):
    iota16 = lax.iota(jnp.int32, 16)
    lane0 = iota16 == 0

    @plsc.parallel_loop(0, 256, unroll=4)
    def _(c):
        h = hist_v[pl.ds(c * 16, 16)]
        tot = jnp.broadcast_to(jnp.sum(h), (16,))
        plsc.store_scatter(totals_v, [jnp.broadcast_to(c, (16,))], tot,
                           mask=lane0)

    sums = [jnp.sum(totals_v[pl.ds(kk * 16, 16)]) for kk in range(16)]
    acc0 = jnp.int32(0)
    kstar = jnp.int32(0)
    found = False
    for kk in range(15, -1, -1):
        s = sums[kk]
        cond = jnp.logical_and(jnp.logical_not(found), (acc0 + s) >= need)
        kstar = jnp.where(cond, kk, kstar)
        acc0 = jnp.where(jnp.logical_or(cond, found), acc0, acc0 + s)
        found = jnp.logical_or(found, cond)

    tv = totals_v[pl.ds(kstar * 16, 16)]
    lane_c, acc1 = _find_cross(tv, acc0, need)
    cstar = kstar * 16 + lane_c

    hf = hist_v[pl.ds(cstar * 16, 16)]
    lane_f, above = _find_cross(hf, acc1, need)
    return cstar * 16 + lane_f, above


def _sc_body(x_hbm, vals_hbm, idx_hbm,
             row0_v, row1_v, hist_v, totals_v, cm_v, ci_v, c2m_v, c2i_v,
             vout_v, iout_v, sem0, sem1):
    wid = lax.axis_index("s") * NC + lax.axis_index("c")
    r0 = wid * ROWS_PER_W

    QTR = COLS // 4
    cps = [
        [pltpu.async_copy(x_hbm.at[r0 + rr, pl.ds(c * QTR, QTR)],
                          (row0_v if rr == 0 else row1_v).at[
                              pl.ds(c * QTR, QTR)],
                          sem0 if rr == 0 else sem1)
         for c in range(4)]
        for rr in range(ROWS_PER_W)
    ]

    iota16 = lax.iota(jnp.int32, 16)
    lane0 = iota16 == 0
    ones = jnp.ones((16,), jnp.int32)
    zeros = jnp.zeros((16,), jnp.int32)

    for rr in range(ROWS_PER_W):
        row_v = row0_v if rr == 0 else row1_v

        @plsc.parallel_loop(0, 256, unroll=8)
        def _(i):
            hist_v[pl.ds(i * 16, 16)] = zeros

        @plsc.parallel_loop(0, CAP // 16, unroll=8)
        def _(i):
            cm_v[pl.ds(i * 16, 16)] = zeros
            ci_v[pl.ds(i * 16, 16)] = zeros

        @plsc.parallel_loop(0, CAP2 // 16)
        def _(i):
            c2m_v[pl.ds(i * 16, 16)] = zeros
            c2i_v[pl.ds(i * 16, 16)] = zeros

        for c in range(4):
            cps[rr][c].wait()

            @plsc.parallel_loop(c * (NVREG // 4), (c + 1) * (NVREG // 4),
                                unroll=8)
            def _(i):
                mu = _mono_u(row_v[pl.ds(i * 16, 16)])
                binf = plsc.bitcast(mu >> jnp.uint32(20), jnp.int32)
                plsc.addupdate_scatter(hist_v, [binf], ones)

        bstar, above1 = _select_bin(hist_v, totals_v, K)
        tlo = jnp.broadcast_to(lax.shift_left(bstar, 20), (16,))
        tlo_u = plsc.bitcast(tlo, jnp.uint32)

        @plsc.parallel_loop(0, NVREG, unroll=8, carry=zeros)
        def p2(i, off_v):
            mu = _mono_u(row_v[pl.ds(i * 16, 16)])
            m = mu >= tlo_u
            pos = off_v + plsc.cumsum(m.astype(jnp.int32)) - 1
            pos = jnp.minimum(pos, jnp.int32(CAP - 1))
            idxv = iota16 + jnp.broadcast_to(i * 16, (16,))
            plsc.store_scatter(cm_v, [pos], plsc.bitcast(mu, jnp.int32),
                               mask=m)
            plsc.store_scatter(ci_v, [pos], idxv, mask=m)
            return off_v + plsc.all_reduce_population_count(m)

        @plsc.parallel_loop(0, 256, unroll=8)
        def _(i):
            hist_v[pl.ds(i * 16, 16)] = zeros

        bstar_u = plsc.bitcast(jnp.broadcast_to(bstar, (16,)), jnp.uint32)

        @plsc.parallel_loop(0, CAP // 16, unroll=4)
        def _(i):
            mu = plsc.bitcast(cm_v[pl.ds(i * 16, 16)], jnp.uint32)
            iseq = (mu >> jnp.uint32(20)) == bstar_u
            bin2 = plsc.bitcast((mu >> jnp.uint32(8)), jnp.int32) & 0xFFF
            plsc.addupdate_scatter(hist_v, [bin2], ones, mask=iseq)

        b2star, _ = _select_bin(hist_v, totals_v, K - above1)
        tlo2 = jnp.broadcast_to(
            lax.shift_left(bstar, 20) | lax.shift_left(b2star, 8), (16,))
        tlo2_u = plsc.bitcast(tlo2, jnp.uint32)

        @plsc.parallel_loop(0, CAP // 16, unroll=4, carry=zeros)
        def p3(i, off_v):
            mono = cm_v[pl.ds(i * 16, 16)]
            mu = plsc.bitcast(mono, jnp.uint32)
            m = mu >= tlo2_u
            pos = off_v + plsc.cumsum(m.astype(jnp.int32)) - 1
            pos = jnp.minimum(pos, jnp.int32(CAP2 - 1))
            plsc.store_scatter(c2m_v, [pos], mono, mask=m)
            plsc.store_scatter(c2i_v, [pos], ci_v[pl.ds(i * 16, 16)], mask=m)
            return off_v + plsc.all_reduce_population_count(m)

        NV2 = CAP2 // 16
        carry0 = tuple(c2m_v[pl.ds(j * 16, 16)] ^ MININT for j in range(NV2))

        @pl.loop(0, K, init_carry=carry0)
        def _(t, vs):
            m01 = jnp.maximum(vs[0], vs[1])
            m23 = jnp.maximum(vs[2], vs[3])
            m45 = jnp.maximum(vs[4], vs[5])
            m67 = jnp.maximum(vs[6], vs[7])
            mx = jnp.maximum(jnp.maximum(m01, m23), jnp.maximum(m45, m67))
            mxs = jnp.broadcast_to(jnp.max(mx), (16,))
            pos = jnp.full((16,), BIG, jnp.int32)
            for j in range(NV2):
                f = plsc.all_reduce_ffs(vs[j] == mxs)
                pj = jnp.where(f < 16, f + jnp.int32(j * 16), BIG)
                pos = jnp.minimum(pos, pj)
            idx_sel = plsc.load_gather(c2i_v, [pos])
            mono_sel = mxs ^ MININT
            dec = mono_sel ^ (~lax.shift_right_arithmetic(mono_sel, 31)
                             | MININT)
            tb = jnp.broadcast_to(t, (16,))
            plsc.store_scatter(vout_v, [tb],
                               plsc.bitcast(dec, jnp.float32), mask=lane0)
            plsc.store_scatter(iout_v, [tb], idx_sel, mask=lane0)
            new_vs = tuple(
                jnp.where((iota16 + jnp.int32(j * 16)) == pos, MININT, vs[j])
                for j in range(NV2))
            return new_vs

        pltpu.sync_copy(vout_v, vals_hbm.at[r0 + rr])
        pltpu.sync_copy(iout_v, idx_hbm.at[r0 + rr])


def kernel(x):
    mesh = plsc.VectorSubcoreMesh(
        core_axis_name="c", subcore_axis_name="s", num_cores=NC,
        num_subcores=NS)
    cp = pltpu.CompilerParams()
    if "needs_layout_passes" in pltpu.CompilerParams.__dataclass_fields__:
        cp = dataclasses.replace(cp, needs_layout_passes=False)
    kern = pl.kernel(
        _sc_body,
        out_type=[
            jax.ShapeDtypeStruct((ROWS, K), jnp.float32),
            jax.ShapeDtypeStruct((ROWS, K), jnp.int32),
        ],
        mesh=mesh,
        scratch_types=[
            pltpu.VMEM((COLS,), jnp.float32),
            pltpu.VMEM((COLS,), jnp.float32),
            pltpu.VMEM((4096,), jnp.int32),
            pltpu.VMEM((256,), jnp.int32),
            pltpu.VMEM((CAP,), jnp.int32),
            pltpu.VMEM((CAP,), jnp.int32),
            pltpu.VMEM((CAP2,), jnp.int32),
            pltpu.VMEM((CAP2,), jnp.int32),
            pltpu.VMEM((K,), jnp.float32),
            pltpu.VMEM((K,), jnp.int32),
            pltpu.SemaphoreType.DMA,
            pltpu.SemaphoreType.DMA,
        ],
        compiler_params=cp,
    )
    vals, idx = kern(x)
    return vals, idx


# speedup vs baseline: 1.0127x; 1.0112x over previous
import dataclasses

import jax
import jax.numpy as jnp
import numpy as np
from jax import lax
from jax.experimental import pallas as pl
from jax.experimental.pallas import tpu as pltpu
from jax.experimental.pallas import tpu_sc as plsc

K = 64
ROWS = 64
COLS = 32768
CAP = 512
CAP2 = 128
NVREG = COLS // 16
NC = 2
NS = 16
ROWS_PER_W = ROWS // (NC * NS)
MININT = np.int32(-2147483648)
BIG = np.int32(2**31 - 1)


def _mono(v):
    b = plsc.bitcast(v, jnp.int32)
    return b ^ (lax.shift_right_arithmetic(b, 31) | MININT)


def _mono_u(v):
    return plsc.bitcast(_mono(v), jnp.uint32)


def _find_cross(h, acc, need):
    rh = lax.rev(h, (0,))
    cs = plsc.cumsum(rh)
    cond = (cs + acc) >= need
    j = jnp.max(plsc.all_reduce_ffs(cond))
    lane = 15 - j
    iota = lax.iota(jnp.int32, 16)
    above = acc + jnp.sum(jnp.where(iota == jnp.broadcast_to(j, (16,)),
                                    cs - rh, 0))
    return lane, above


def _select_bin(hist_v, totals_v, need):
    iota16 = lax.iota(jnp.int32, 16)
    lane0 = iota16 == 0

    @plsc.parallel_loop(0, 256, unroll=4)
    def _(c):
        h = hist_v[pl.ds(c * 16, 16)]
        tot = jnp.broadcast_to(jnp.sum(h), (16,))
        plsc.store_scatter(totals_v, [jnp.broadcast_to(c, (16,))], tot,
                           mask=lane0)

    def cscan(j, carry):
        acc, kstar, found = carry
        kk = 15 - j
        s = jnp.sum(totals_v[pl.ds(kk * 16, 16)])
        cond = jnp.logical_and(jnp.logical_not(found), (acc + s) >= need)
        kstar = jnp.where(cond, kk, kstar)
        acc = jnp.where(jnp.logical_or(cond, found), acc, acc + s)
        found = jnp.logical_or(found, cond)
        return acc, kstar, found

    acc0, kstar, _ = lax.fori_loop(
        0, 16, cscan, (jnp.int32(0), jnp.int32(0), False))

    tv = totals_v[pl.ds(kstar * 16, 16)]
    lane_c, acc1 = _find_cross(tv, acc0, need)
    cstar = kstar * 16 + lane_c

    hf = hist_v[pl.ds(cstar * 16, 16)]
    lane_f, above = _find_cross(hf, acc1, need)
    return cstar * 16 + lane_f, above


def _sc_body(x_hbm, vals_hbm, idx_hbm,
             row0_v, row1_v, hist_v, totals_v, cm_v, ci_v, c2m_v, c2i_v,
             vout_v, iout_v, sem0, sem1):
    wid = lax.axis_index("s") * NC + lax.axis_index("c")
    r0 = wid * ROWS_PER_W

    QTR = COLS // 4
    cps = [
        [pltpu.async_copy(x_hbm.at[r0 + rr, pl.ds(c * QTR, QTR)],
                          (row0_v if rr == 0 else row1_v).at[
                              pl.ds(c * QTR, QTR)],
                          sem0 if rr == 0 else sem1)
         for c in range(4)]
        for rr in range(ROWS_PER_W)
    ]

    iota16 = lax.iota(jnp.int32, 16)
    lane0 = iota16 == 0
    ones = jnp.ones((16,), jnp.int32)
    zeros = jnp.zeros((16,), jnp.int32)

    for rr in range(ROWS_PER_W):
        row_v = row0_v if rr == 0 else row1_v

        @plsc.parallel_loop(0, 256, unroll=8)
        def _(i):
            hist_v[pl.ds(i * 16, 16)] = zeros

        @plsc.parallel_loop(0, CAP // 16, unroll=8)
        def _(i):
            cm_v[pl.ds(i * 16, 16)] = zeros
            ci_v[pl.ds(i * 16, 16)] = zeros

        @plsc.parallel_loop(0, CAP2 // 16)
        def _(i):
            c2m_v[pl.ds(i * 16, 16)] = zeros
            c2i_v[pl.ds(i * 16, 16)] = zeros

        for c in range(4):
            cps[rr][c].wait()

            @plsc.parallel_loop(c * (NVREG // 4), (c + 1) * (NVREG // 4),
                                unroll=8)
            def _(i):
                mu = _mono_u(row_v[pl.ds(i * 16, 16)])
                binf = plsc.bitcast(mu >> jnp.uint32(20), jnp.int32)
                plsc.addupdate_scatter(hist_v, [binf], ones)

        bstar, above1 = _select_bin(hist_v, totals_v, K)
        tlo = jnp.broadcast_to(lax.shift_left(bstar, 20), (16,))
        tlo_u = plsc.bitcast(tlo, jnp.uint32)

        @plsc.parallel_loop(0, NVREG, unroll=8, carry=zeros)
        def p2(i, off_v):
            mu = _mono_u(row_v[pl.ds(i * 16, 16)])
            m = mu >= tlo_u
            pos = off_v + plsc.cumsum(m.astype(jnp.int32)) - 1
            pos = jnp.minimum(pos, jnp.int32(CAP - 1))
            idxv = iota16 + jnp.broadcast_to(i * 16, (16,))
            plsc.store_scatter(cm_v, [pos], plsc.bitcast(mu, jnp.int32),
                               mask=m)
            plsc.store_scatter(ci_v, [pos], idxv, mask=m)
            return off_v + plsc.all_reduce_population_count(m)

        @plsc.parallel_loop(0, 256, unroll=8)
        def _(i):
            hist_v[pl.ds(i * 16, 16)] = zeros

        bstar_u = plsc.bitcast(jnp.broadcast_to(bstar, (16,)), jnp.uint32)

        @plsc.parallel_loop(0, CAP // 16, unroll=4)
        def _(i):
            mu = plsc.bitcast(cm_v[pl.ds(i * 16, 16)], jnp.uint32)
            iseq = (mu >> jnp.uint32(20)) == bstar_u
            bin2 = plsc.bitcast((mu >> jnp.uint32(8)), jnp.int32) & 0xFFF
            plsc.addupdate_scatter(hist_v, [bin2], ones, mask=iseq)

        b2star, _ = _select_bin(hist_v, totals_v, K - above1)
        tlo2 = jnp.broadcast_to(
            lax.shift_left(bstar, 20) | lax.shift_left(b2star, 8), (16,))
        tlo2_u = plsc.bitcast(tlo2, jnp.uint32)

        @plsc.parallel_loop(0, CAP // 16, unroll=4, carry=zeros)
        def p3(i, off_v):
            mono = cm_v[pl.ds(i * 16, 16)]
            mu = plsc.bitcast(mono, jnp.uint32)
            m = mu >= tlo2_u
            pos = off_v + plsc.cumsum(m.astype(jnp.int32)) - 1
            pos = jnp.minimum(pos, jnp.int32(CAP2 - 1))
            plsc.store_scatter(c2m_v, [pos], mono, mask=m)
            plsc.store_scatter(c2i_v, [pos], ci_v[pl.ds(i * 16, 16)], mask=m)
            return off_v + plsc.all_reduce_population_count(m)

        NV2 = CAP2 // 16
        carry0 = tuple(c2m_v[pl.ds(j * 16, 16)] ^ MININT for j in range(NV2))

        @pl.loop(0, K, init_carry=carry0)
        def _(t, vs):
            m01 = jnp.maximum(vs[0], vs[1])
            m23 = jnp.maximum(vs[2], vs[3])
            m45 = jnp.maximum(vs[4], vs[5])
            m67 = jnp.maximum(vs[6], vs[7])
            mx = jnp.maximum(jnp.maximum(m01, m23), jnp.maximum(m45, m67))
            mxs = jnp.broadcast_to(jnp.max(mx), (16,))
            pos = jnp.full((16,), BIG, jnp.int32)
            for j in range(NV2):
                f = plsc.all_reduce_ffs(vs[j] == mxs)
                pj = jnp.where(f < 16, f + jnp.int32(j * 16), BIG)
                pos = jnp.minimum(pos, pj)
            idx_sel = plsc.load_gather(c2i_v, [pos])
            mono_sel = mxs ^ MININT
            dec = mono_sel ^ (~lax.shift_right_arithmetic(mono_sel, 31)
                             | MININT)
            tb = jnp.broadcast_to(t, (16,))
            plsc.store_scatter(vout_v, [tb],
                               plsc.bitcast(dec, jnp.float32), mask=lane0)
            plsc.store_scatter(iout_v, [tb], idx_sel, mask=lane0)
            new_vs = tuple(
                jnp.where((iota16 + jnp.int32(j * 16)) == pos, MININT, vs[j])
                for j in range(NV2))
            return new_vs

        pltpu.sync_copy(vout_v, vals_hbm.at[r0 + rr])
        pltpu.sync_copy(iout_v, idx_hbm.at[r0 + rr])


def kernel(x):
    mesh = plsc.VectorSubcoreMesh(
        core_axis_name="c", subcore_axis_name="s", num_cores=NC,
        num_subcores=NS)
    cp = pltpu.CompilerParams()
    if "needs_layout_passes" in pltpu.CompilerParams.__dataclass_fields__:
        cp = dataclasses.replace(cp, needs_layout_passes=False)
    kern = pl.kernel(
        _sc_body,
        out_type=[
            jax.ShapeDtypeStruct((ROWS, K), jnp.float32),
            jax.ShapeDtypeStruct((ROWS, K), jnp.int32),
        ],
        mesh=mesh,
        scratch_types=[
            pltpu.VMEM((COLS,), jnp.float32),
            pltpu.VMEM((COLS,), jnp.float32),
            pltpu.VMEM((4096,), jnp.int32),
            pltpu.VMEM((256,), jnp.int32),
            pltpu.VMEM((CAP,), jnp.int32),
            pltpu.VMEM((CAP,), jnp.int32),
            pltpu.VMEM((CAP2,), jnp.int32),
            pltpu.VMEM((CAP2,), jnp.int32),
            pltpu.VMEM((K,), jnp.float32),
            pltpu.VMEM((K,), jnp.int32),
            pltpu.SemaphoreType.DMA,
            pltpu.SemaphoreType.DMA,
        ],
        compiler_params=cp,
    )
    vals, idx = kern(x)
    return vals, idx
